# baseline (speedup 1.0000x reference)
import functools

import jax
import jax.numpy as jnp
from jax import lax
from jax.experimental import pallas as pl
from jax.experimental.pallas import tpu as pltpu
from jax.experimental.pallas import tpu_sc as plsc

N = 160000
D = 256
C = 1000
CP = 1024
CHUNK = 128
NCHUNKS = N // CHUNK
NCORES = 2
NSUB = 16
NW = NCORES * NSUB

BR = 2000
NB = N // BR


def _centers_body(feat_ref, lab_ref, cen_ref, cnt_ref, sums_acc, cnt_acc):
    i = pl.program_id(0)

    @pl.when(i == 0)
    def _():
        sums_acc[...] = jnp.zeros((CP, D), jnp.float32)
        cnt_acc[...] = jnp.zeros((CP, 128), jnp.float32)

    lab = lab_ref[0, 0, :]
    fbf = feat_ref[...].astype(jnp.bfloat16)
    b0 = jnp.minimum(jnp.min(lab) // 8 * 8, CP - 256)

    def window(k):
        base = pl.multiple_of(b0 + 256 * k, 8)
        rel = lab - base

        @pl.when(jnp.any((rel >= 0) & (rel < 256)))
        def _():
            oh = (rel[:, None] ==
                  lax.broadcasted_iota(jnp.int32, (BR, 256), 1)
                  ).astype(jnp.float32)
            part = lax.dot_general(
                oh.astype(jnp.bfloat16), fbf, (((0,), (0,)), ((), ())),
                preferred_element_type=jnp.float32)
            sums_acc[pl.ds(base, 256), :] += part
            cnt_acc[pl.ds(base, 256), :] += jnp.sum(oh, axis=0)[:, None]

    for k in range(4):
        window(k)

    @pl.when(i == NB - 1)
    def _():
        cnt = cnt_acc[...][:, 0:1]
        safe = jnp.where(cnt > 0, cnt, 1.0)
        cen = jnp.where(cnt > 0, sums_acc[...] / safe, 0.0)
        lo = lax.bitcast_convert_type(
            cen[:, :128].astype(jnp.bfloat16), jnp.uint16).astype(jnp.uint32)
        hi = lax.bitcast_convert_type(
            cen[:, 128:].astype(jnp.bfloat16), jnp.uint16).astype(jnp.uint32)
        cen_ref[...] = lax.bitcast_convert_type((hi << 16) | lo, jnp.float32)
        cnt_ref[...] = cnt_acc[...]


def _compute_centers(features, lab3d):
    return pl.pallas_call(
        _centers_body,
        grid=(NB,),
        in_specs=[
            pl.BlockSpec((BR, D), lambda i: (i, 0)),
            pl.BlockSpec((1, 1, BR), lambda i: (i, 0, 0)),
        ],
        out_specs=[
            pl.BlockSpec((CP, 128), lambda i: (0, 0)),
            pl.BlockSpec((CP, 128), lambda i: (0, 0)),
        ],
        out_shape=[
            jax.ShapeDtypeStruct((CP, 128), jnp.float32),
            jax.ShapeDtypeStruct((CP, 128), jnp.float32),
        ],
        scratch_shapes=[
            pltpu.VMEM((CP, D), jnp.float32),
            pltpu.VMEM((CP, 128), jnp.float32),
        ],
    )(features, lab3d)


NSLICE = 5
SROWS = N // NSLICE
SCHUNKS = SROWS // CHUNK
STAGE = 8


def _gather_body(cbase, centers, lab1d, out, bufa, bufb, idxb,
                 sg1, sg2, sg3, sg4, sw1, sw2, sw3):
    cid = lax.axis_index("c")
    sid = lax.axis_index("s")
    wid = sid * NCORES + cid
    c0 = (wid * SCHUNKS) // NW
    lrow = c0 * CHUNK

    pltpu.sync_copy(
        lab1d.at[pl.ds((cbase + c0) * CHUNK, STAGE * CHUNK)], idxb)

    g1 = pltpu.async_copy(centers.at[idxb.at[pl.ds(0, 3 * CHUNK)]],
                          bufa.at[pl.ds(0, 3 * CHUNK)], sg1)
    g2 = pltpu.async_copy(centers.at[idxb.at[pl.ds(3 * CHUNK, 3 * CHUNK)]],
                          bufb, sg2)
    g3 = pltpu.async_copy(centers.at[idxb.at[pl.ds(6 * CHUNK, CHUNK)]],
                          bufa.at[pl.ds(3 * CHUNK, CHUNK)], sg3)
    g1.wait()
    w1 = pltpu.async_copy(bufa.at[pl.ds(0, 3 * CHUNK)],
                          out.at[pl.ds(lrow, 3 * CHUNK)], sw1)
    g2.wait()
    w2 = pltpu.async_copy(bufb, out.at[pl.ds(lrow + 3 * CHUNK, 3 * CHUNK)], sw2)
    g3.wait()
    w3 = pltpu.async_copy(bufa.at[pl.ds(3 * CHUNK, CHUNK)],
                          out.at[pl.ds(lrow + 6 * CHUNK, CHUNK)], sw3)
    w1.wait()
    g4 = pltpu.async_copy(centers.at[idxb.at[pl.ds(7 * CHUNK, CHUNK)]],
                          bufa.at[pl.ds(0, CHUNK)], sg4)
    g4.wait()
    w4 = pltpu.async_copy(bufa.at[pl.ds(0, CHUNK)],
                          out.at[pl.ds(lrow + 7 * CHUNK, CHUNK)], sw1)
    w2.wait()
    w3.wait()
    w4.wait()


@functools.lru_cache(maxsize=None)
def _gather_kernel(slice_idx):
    mesh = plsc.VectorSubcoreMesh(core_axis_name="c", subcore_axis_name="s")
    return pl.kernel(
        functools.partial(_gather_body, slice_idx * SCHUNKS),
        out_type=jax.ShapeDtypeStruct((SROWS, 128), jnp.float32),
        mesh=mesh,
        scratch_types=(
            [
                pltpu.VMEM((4 * CHUNK, 128), jnp.float32),
                pltpu.VMEM((3 * CHUNK, 128), jnp.float32),
                pltpu.VMEM((STAGE * CHUNK,), jnp.int32),
            ]
            + [pltpu.SemaphoreType.DMA] * 7
        ),
    )


NBS = SROWS // BR


def _dist_body(feat_ref, gath_ref, lab_ref, part_ref, acc_ref):
    i = pl.program_id(0)

    @pl.when(i == 0)
    def _():
        acc_ref[...] = jnp.zeros((CP, 128), jnp.float32)

    gp = lax.bitcast_convert_type(gath_ref[...], jnp.uint32)
    clo = lax.bitcast_convert_type(
        (gp & 0xFFFF).astype(jnp.uint16), jnp.bfloat16).astype(jnp.float32)
    chi = lax.bitcast_convert_type(
        (gp >> 16).astype(jnp.uint16), jnp.bfloat16).astype(jnp.float32)
    f = feat_ref[...]
    dlo = f[:, :128] - clo + 1e-6
    dhi = f[:, 128:] - chi + 1e-6
    dist = jnp.sqrt(jnp.sum(dlo * dlo, axis=1) +
                    jnp.sum(dhi * dhi, axis=1))[:, None]
    lab = lab_ref[0, 0, :]
    onehot = (lab[:, None] ==
              lax.broadcasted_iota(jnp.int32, (BR, CP), 1)).astype(jnp.float32)
    acc_ref[...] += lax.dot_general(
        onehot, dist, (((0,), (0,)), ((), ())),
        preferred_element_type=jnp.float32)

    @pl.when(i == NBS - 1)
    def _():
        part_ref[...] = acc_ref[...]


def _dist_partial(s, features, gathered, lab3d):
    return pl.pallas_call(
        _dist_body,
        grid=(NBS,),
        in_specs=[
            pl.BlockSpec((BR, D), lambda i, s=s: (s * NBS + i, 0)),
            pl.BlockSpec((BR, 128), lambda i: (i, 0)),
            pl.BlockSpec((1, 1, BR), lambda i, s=s: (s * NBS + i, 0, 0)),
        ],
        out_specs=pl.BlockSpec((CP, 128), lambda i: (0, 0)),
        out_shape=jax.ShapeDtypeStruct((CP, 128), jnp.float32),
        scratch_shapes=[pltpu.VMEM((CP, 128), jnp.float32)],
    )(features, gathered, lab3d)


def _final_body(parts_ref, cnt_ref, loss_ref):
    cnt = cnt_ref[...][:, 0]
    ds = jnp.sum(parts_ref[...][:, 0].reshape(NSLICE, CP), axis=0)
    safe = jnp.where(cnt > 0, cnt, 1.0)
    loss_ref[...] = jnp.sum(jnp.where(cnt > 0, ds / safe, 0.0))[None, None]


def _final_loss(parts, cnt):
    return pl.pallas_call(
        _final_body,
        out_shape=jax.ShapeDtypeStruct((1, 1), jnp.float32),
    )(parts, cnt)


def kernel(features, labels):
    lab32 = labels.astype(jnp.int32)
    lab3d = lab32.reshape(NB, 1, BR)
    centers, cnt = _compute_centers(features, lab3d)
    parts = []
    for s in range(NSLICE):
        gathered = _gather_kernel(s)(centers, lab32)
        parts.append(_dist_partial(s, features, gathered, lab3d))
    loss = _final_loss(jnp.concatenate(parts, axis=0), cnt)
    return loss[0, 0]


# speedup vs baseline: 1.0784x; 1.0784x over previous
import functools

import jax
import jax.numpy as jnp
from jax import lax
from jax.experimental import pallas as pl
from jax.experimental.pallas import tpu as pltpu
from jax.experimental.pallas import tpu_sc as plsc

N = 160000
D = 256
C = 1000
CP = 1024
CHUNK = 128
NCHUNKS = N // CHUNK
NCORES = 2
NSUB = 16
NW = NCORES * NSUB

BR = 2000
NB = N // BR


def _centers_body(feat_ref, lab_ref, cen_ref, cnt_ref, sums_acc, cnt_acc):
    i = pl.program_id(0)

    @pl.when(i == 0)
    def _():
        sums_acc[...] = jnp.zeros((CP, D), jnp.float32)
        cnt_acc[...] = jnp.zeros((CP, 128), jnp.float32)

    lab = lab_ref[0, 0, :]
    fbf = feat_ref[...].astype(jnp.bfloat16)
    b0 = jnp.minimum(jnp.min(lab) // 8 * 8, CP - 256)

    def window(k):
        base = pl.multiple_of(b0 + 256 * k, 8)
        rel = lab - base

        @pl.when(jnp.any((rel >= 0) & (rel < 256)))
        def _():
            oh = (rel[:, None] ==
                  lax.broadcasted_iota(jnp.int32, (BR, 256), 1)
                  ).astype(jnp.float32)
            part = lax.dot_general(
                oh.astype(jnp.bfloat16), fbf, (((0,), (0,)), ((), ())),
                preferred_element_type=jnp.float32)
            sums_acc[pl.ds(base, 256), :] += part
            cnt_acc[pl.ds(base, 256), :] += jnp.sum(oh, axis=0)[:, None]

    for k in range(4):
        window(k)

    @pl.when(i == NB - 1)
    def _():
        cnt = cnt_acc[...][:, 0:1]
        safe = jnp.where(cnt > 0, cnt, 1.0)
        cen = jnp.where(cnt > 0, sums_acc[...] / safe, 0.0)
        lo = lax.bitcast_convert_type(
            cen[:, :128].astype(jnp.bfloat16), jnp.uint16).astype(jnp.uint32)
        hi = lax.bitcast_convert_type(
            cen[:, 128:].astype(jnp.bfloat16), jnp.uint16).astype(jnp.uint32)
        cen_ref[...] = lax.bitcast_convert_type((hi << 16) | lo, jnp.float32)
        cnt_ref[...] = cnt_acc[...]


def _compute_centers(features, lab3d):
    return pl.pallas_call(
        _centers_body,
        grid=(NB,),
        in_specs=[
            pl.BlockSpec((BR, D), lambda i: (i, 0)),
            pl.BlockSpec((1, 1, BR), lambda i: (i, 0, 0)),
        ],
        out_specs=[
            pl.BlockSpec((CP, 128), lambda i: (0, 0)),
            pl.BlockSpec((CP, 128), lambda i: (0, 0)),
        ],
        out_shape=[
            jax.ShapeDtypeStruct((CP, 128), jnp.float32),
            jax.ShapeDtypeStruct((CP, 128), jnp.float32),
        ],
        scratch_shapes=[
            pltpu.VMEM((CP, D), jnp.float32),
            pltpu.VMEM((CP, 128), jnp.float32),
        ],
    )(features, lab3d)


NSLICE = 5
SROWS = N // NSLICE
SCHUNKS = SROWS // CHUNK
STAGE = 8


def _gather_body(cbase, centers, lab1d, out, bufa, bufb, idxb,
                 sg1, sg2, sg3, sg4, sw1, sw2, sw3):
    cid = lax.axis_index("c")
    sid = lax.axis_index("s")
    wid = sid * NCORES + cid
    c0 = (wid * SCHUNKS) // NW
    lrow = c0 * CHUNK

    pltpu.sync_copy(
        lab1d.at[pl.ds((cbase + c0) * CHUNK, STAGE * CHUNK)], idxb)

    g1 = pltpu.async_copy(centers.at[idxb.at[pl.ds(0, 3 * CHUNK)]],
                          bufa.at[pl.ds(0, 3 * CHUNK)], sg1)
    g2 = pltpu.async_copy(centers.at[idxb.at[pl.ds(3 * CHUNK, 3 * CHUNK)]],
                          bufb, sg2)
    g3 = pltpu.async_copy(centers.at[idxb.at[pl.ds(6 * CHUNK, CHUNK)]],
                          bufa.at[pl.ds(3 * CHUNK, CHUNK)], sg3)
    g1.wait()
    w1 = pltpu.async_copy(bufa.at[pl.ds(0, 3 * CHUNK)],
                          out.at[pl.ds(lrow, 3 * CHUNK)], sw1)
    g2.wait()
    w2 = pltpu.async_copy(bufb, out.at[pl.ds(lrow + 3 * CHUNK, 3 * CHUNK)], sw2)
    g3.wait()
    w3 = pltpu.async_copy(bufa.at[pl.ds(3 * CHUNK, CHUNK)],
                          out.at[pl.ds(lrow + 6 * CHUNK, CHUNK)], sw3)
    w1.wait()
    g4 = pltpu.async_copy(centers.at[idxb.at[pl.ds(7 * CHUNK, CHUNK)]],
                          bufa.at[pl.ds(0, CHUNK)], sg4)
    g4.wait()
    w4 = pltpu.async_copy(bufa.at[pl.ds(0, CHUNK)],
                          out.at[pl.ds(lrow + 7 * CHUNK, CHUNK)], sw1)
    w2.wait()
    w3.wait()
    w4.wait()


@functools.lru_cache(maxsize=None)
def _gather_kernel(slice_idx):
    mesh = plsc.VectorSubcoreMesh(core_axis_name="c", subcore_axis_name="s")
    return pl.kernel(
        functools.partial(_gather_body, slice_idx * SCHUNKS),
        out_type=jax.ShapeDtypeStruct((SROWS, 128), jnp.float32),
        mesh=mesh,
        scratch_types=(
            [
                pltpu.VMEM((4 * CHUNK, 128), jnp.float32),
                pltpu.VMEM((3 * CHUNK, 128), jnp.float32),
                pltpu.VMEM((STAGE * CHUNK,), jnp.int32),
            ]
            + [pltpu.SemaphoreType.DMA] * 7
        ),
    )


NBS = SROWS // BR


def _dist_body(feat_ref, gath_ref, lab_ref, part_ref, acc_ref):
    i = pl.program_id(0)

    @pl.when(i == 0)
    def _():
        acc_ref[...] = jnp.zeros((CP, 128), jnp.float32)

    gp = lax.bitcast_convert_type(gath_ref[...], jnp.uint32)
    clo = lax.bitcast_convert_type(
        (gp & 0xFFFF).astype(jnp.uint16), jnp.bfloat16).astype(jnp.float32)
    chi = lax.bitcast_convert_type(
        (gp >> 16).astype(jnp.uint16), jnp.bfloat16).astype(jnp.float32)
    f = feat_ref[...]
    dlo = f[:, :128] - clo + 1e-6
    dhi = f[:, 128:] - chi + 1e-6
    dist = jnp.sqrt(jnp.sum(dlo * dlo, axis=1) +
                    jnp.sum(dhi * dhi, axis=1))[:, None]
    lab = lab_ref[0, 0, :]
    b0 = jnp.minimum(jnp.min(lab) // 8 * 8, CP - 256)

    for k in range(4):
        base = pl.multiple_of(b0 + 256 * k, 8)
        rel = lab - base

        @pl.when(jnp.any((rel >= 0) & (rel < 256)))
        def _():
            oh = (rel[:, None] ==
                  lax.broadcasted_iota(jnp.int32, (BR, 256), 1)
                  ).astype(jnp.float32)
            part = lax.dot_general(
                oh, dist, (((0,), (0,)), ((), ())),
                preferred_element_type=jnp.float32)
            acc_ref[pl.ds(base, 256), :] += part

    @pl.when(i == NBS - 1)
    def _():
        part_ref[...] = acc_ref[...]


def _dist_partial(s, features, gathered, lab3d):
    return pl.pallas_call(
        _dist_body,
        grid=(NBS,),
        in_specs=[
            pl.BlockSpec((BR, D), lambda i, s=s: (s * NBS + i, 0)),
            pl.BlockSpec((BR, 128), lambda i: (i, 0)),
            pl.BlockSpec((1, 1, BR), lambda i, s=s: (s * NBS + i, 0, 0)),
        ],
        out_specs=pl.BlockSpec((CP, 128), lambda i: (0, 0)),
        out_shape=jax.ShapeDtypeStruct((CP, 128), jnp.float32),
        scratch_shapes=[pltpu.VMEM((CP, 128), jnp.float32)],
    )(features, gathered, lab3d)


def _final_body(parts_ref, cnt_ref, loss_ref):
    cnt = cnt_ref[...][:, 0]
    ds = jnp.sum(parts_ref[...][:, 0].reshape(NSLICE, CP), axis=0)
    safe = jnp.where(cnt > 0, cnt, 1.0)
    loss_ref[...] = jnp.sum(jnp.where(cnt > 0, ds / safe, 0.0))[None, None]


def _final_loss(parts, cnt):
    return pl.pallas_call(
        _final_body,
        out_shape=jax.ShapeDtypeStruct((1, 1), jnp.float32),
    )(parts, cnt)


def kernel(features, labels):
    lab32 = labels.astype(jnp.int32)
    lab3d = lab32.reshape(NB, 1, BR)
    centers, cnt = _compute_centers(features, lab3d)
    parts = []
    for s in range(NSLICE):
        gathered = _gather_kernel(s)(centers, lab32)
        parts.append(_dist_partial(s, features, gathered, lab3d))
    loss = _final_loss(jnp.concatenate(parts, axis=0), cnt)
    return loss[0, 0]
